# baseline (speedup 1.0000x reference)
import functools
import numpy as np
import jax
import jax.numpy as jnp
from jax.experimental import pallas as pl
from jax.experimental.pallas import tpu as pltpu

IN_C, OUT_C, KSIZE, IMG = 3, 8, 5, 32
POOL_HW = 14
FC_OUT = 10
F_PAD = 16
TB = 256
NM = 224
NK = 160
FLAT = IN_C * IMG * IMG


def _chan_weights(Wc):
    m = np.arange(NM)
    xpar = m // 112
    j = (m // OUT_C) % POOL_HW
    o = m % OUT_C
    k = np.arange(NK)
    yloc = k // IMG
    xin = k % IMG
    kx = xin[None, :] - (2 * j + xpar)[:, None]
    valid = (kx >= 0) & (kx < KSIZE)
    ws = []
    for c in range(IN_C):
        src = ((o[:, None] * IN_C + c) * KSIZE + yloc[None, :]) * KSIZE \
            + np.clip(kx, 0, KSIZE - 1)
        wb = jnp.where(jnp.asarray(valid), Wc.reshape(-1)[jnp.asarray(src)], 0.0)
        ws.append(wb.astype(jnp.bfloat16))
    return jnp.stack(ws)


def _fc_weight(Wf):
    w4 = Wf.reshape(FC_OUT, OUT_C, POOL_HW, POOL_HW)
    w4 = jnp.transpose(w4, (2, 0, 3, 1)).reshape(POOL_HW, FC_OUT, 112)
    w4 = jnp.pad(w4, ((0, 0), (0, F_PAD - FC_OUT), (0, 0)))
    return w4.astype(jnp.bfloat16)


_DN = (((1,), (1,)), ((), ()))


def _net_kernel(x_ref, w_ref, bc_ref, wf_ref, bf_ref, out_ref, xb_ref):
    xb_ref[...] = x_ref[0].astype(jnp.bfloat16)
    w_c = [w_ref[0], w_ref[1], w_ref[2]]
    bc = bc_ref[...]
    accs = [jnp.zeros(out_ref.shape, jnp.float32) for _ in range(2)]
    for i in range(POOL_HW):
        rs = []
        for ypar in range(2):
            y0 = (2 * i + ypar) * IMG
            r = None
            for c in range(IN_C):
                xs = xb_ref[:, c * IMG * IMG + y0:c * IMG * IMG + y0 + NK]
                d = jax.lax.dot_general(w_c[c], xs, _DN,
                                        preferred_element_type=jnp.float32)
                r = d if r is None else r + d
            rs.append(r)
        r0, r1 = rs
        m = jnp.maximum(jnp.maximum(r0[:112], r0[112:]),
                        jnp.maximum(r1[:112], r1[112:]))
        a = jnp.maximum(m + bc, 0.0).astype(jnp.bfloat16)
        accs[i % 2] = accs[i % 2] + jnp.dot(
            wf_ref[i], a, preferred_element_type=jnp.float32)
    out_ref[...] = accs[0] + accs[1] + bf_ref[...]


@jax.jit
def _forward(x, Wc, bc, Wf, bf):
    B = x.shape[0]
    grid = pl.cdiv(B, TB)
    Bp = grid * TB
    if Bp != B:
        x = jnp.pad(x, ((0, Bp - B), (0, 0), (0, 0), (0, 0)))
    xr = x.reshape(grid, TB, FLAT)

    wn = _chan_weights(Wc)
    bcr = jnp.tile(bc.astype(jnp.float32), POOL_HW).reshape(112, 1)
    wf_r = _fc_weight(Wf)
    bf_col = jnp.pad(bf.astype(jnp.float32), (0, F_PAD - FC_OUT)).reshape(F_PAD, 1)

    flops = 2 * Bp * POOL_HW * (6 * NM * NK + F_PAD * 112)
    bytes_accessed = grid * TB * FLAT * 4 + 3 * NM * NK * 2 + F_PAD * Bp * 4

    out = pl.pallas_call(
        _net_kernel,
        out_shape=jax.ShapeDtypeStruct((F_PAD, Bp), jnp.float32),
        grid=(grid,),
        in_specs=[
            pl.BlockSpec((1, TB, FLAT), lambda b: (b, 0, 0)),
            pl.BlockSpec((IN_C, NM, NK), lambda b: (0, 0, 0)),
            pl.BlockSpec((112, 1), lambda b: (0, 0)),
            pl.BlockSpec((POOL_HW, F_PAD, 112), lambda b: (0, 0, 0)),
            pl.BlockSpec((F_PAD, 1), lambda b: (0, 0)),
        ],
        out_specs=pl.BlockSpec((F_PAD, TB), lambda b: (0, b)),
        scratch_shapes=[pltpu.VMEM((TB, FLAT), jnp.bfloat16)],
        compiler_params=pltpu.CompilerParams(
            dimension_semantics=("parallel",),
        ),
        cost_estimate=pl.CostEstimate(flops=int(flops), transcendentals=0,
                                      bytes_accessed=int(bytes_accessed)),
    )(xr, wn, bcr, wf_r, bf_col)
    return jnp.transpose(out[:FC_OUT, :B])


def kernel(x, Wc, bc, Wf, bf):
    return _forward(x, Wc, bc, Wf, bf)
